# Initial kernel scaffold; baseline (speedup 1.0000x reference)
import jax
import jax.numpy as jnp
from jax.experimental import pallas as pl


def kernel(h, x, pos, edge_index_gate, edge_index_cand, Wg, bg, Wc, bc):
    raise NotImplementedError("write your pallas kernel here")


# speedup vs baseline: 1.0591x; 1.0591x over previous
import functools

import jax
import jax.numpy as jnp
from jax import lax
from jax.experimental import pallas as pl
from jax.experimental.pallas import tpu as pltpu
from jax.experimental.pallas import tpu_sc as plsc

_N = 10000
_E = 320000
_F_IN = 128
_F_OUT = 128
_NW = 32
_NPT = 313
_NPAD = _NW * _NPT
_NEG = float(jnp.finfo(jnp.float32).min)


def _make_segmax(F: int):
    C = 2000
    G = 64
    NCH = _E // C
    PK = C + 80
    Fv = F // 16

    mesh = plsc.VectorSubcoreMesh(core_axis_name="c", subcore_axis_name="s")

    def body(b_hbm, dst_hbm, src_hbm, m_hbm, dstb, srcb, pks, pkd, rows, acc, sem):
        wid = lax.axis_index("s") * 2 + lax.axis_index("c")
        n0 = wid * _NPT

        def ini_acc(i, _):
            acc[pl.ds(i * 16, 16)] = jnp.full((16,), _NEG, jnp.float32)
            return 0
        lax.fori_loop(0, _NPT * Fv, ini_acc, 0)

        def ini_pk(i, _):
            pks[pl.ds(i * 16, 16)] = jnp.zeros((16,), jnp.int32)
            return 0
        lax.fori_loop(0, PK // 16, ini_pk, 0)

        def chunk(c, _):
            pltpu.sync_copy(dst_hbm.at[pl.ds(c * C, C)], dstb)
            pltpu.sync_copy(src_hbm.at[pl.ds(c * C, C)], srcb)

            def filt(k, cnt):
                d = dstb[pl.ds(k * 16, 16)]
                s = srcb[pl.ds(k * 16, 16)]
                msk = (d >= n0) & (d < n0 + _NPT)
                pc = plsc.cumsum(msk.astype(jnp.int32))
                idx = cnt + pc - 1
                plsc.store_scatter(pks, [idx], s, mask=msk)
                plsc.store_scatter(pkd, [idx], d - n0, mask=msk)
                return cnt + jnp.max(pc)
            cnt = lax.fori_loop(0, C // 16, filt, 0)

            def sub(sb, _):
                base = sb * G
                pltpu.async_copy(b_hbm.at[pks.at[pl.ds(base, G)]], rows, sem).wait()
                m = jnp.minimum(G, cnt - base)

                def upd(i, _):
                    ro = pkd[pl.ds(base + i, 16)][0] * F
                    for j in range(Fv):
                        a = acc[pl.ds(ro + j * 16, 16)]
                        r = rows[i, pl.ds(j * 16, 16)]
                        acc[pl.ds(ro + j * 16, 16)] = jnp.maximum(a, r)
                    return 0
                lax.fori_loop(0, m, upd, 0)
                return 0
            lax.fori_loop(0, (cnt + G - 1) // G, sub, 0)
            return 0
        lax.fori_loop(0, NCH, chunk, 0)

        pltpu.sync_copy(acc, m_hbm.at[pl.ds(n0 * F, _NPT * F)])

    return pl.kernel(
        body,
        mesh=mesh,
        compiler_params=pltpu.CompilerParams(needs_layout_passes=False),
        out_type=jax.ShapeDtypeStruct((_NPAD * F,), jnp.float32),
        scratch_types=[
            pltpu.VMEM((C,), jnp.int32),
            pltpu.VMEM((C,), jnp.int32),
            pltpu.VMEM((PK,), jnp.int32),
            pltpu.VMEM((PK,), jnp.int32),
            pltpu.VMEM((G, F), jnp.float32),
            pltpu.VMEM((_NPT * F,), jnp.float32),
            pltpu.SemaphoreType.DMA,
        ],
    )


_segmax_gate = _make_segmax(2 * _F_OUT)
_segmax_cand = _make_segmax(_F_OUT)


_BLK = 2000
_GRID = _N // _BLK


def _pos_term(pos_ref, w3):
    return (pos_ref[:, 0:1] * w3[0:1, :]
            + pos_ref[:, 1:2] * w3[1:2, :]
            + pos_ref[:, 2:3] * w3[2:3, :])


def _tc1_body(xh_ref, pos_ref, w1_ref, w2_ref, w3_ref, b_ref, a_out, b_out):
    xh = xh_ref[...]
    w2 = w2_ref[...]
    pw = _pos_term(pos_ref, w3_ref[...])
    b_out[...] = jnp.dot(xh, w2, preferred_element_type=jnp.float32) + pw
    a_out[...] = (jnp.dot(xh, w1_ref[...] - w2, preferred_element_type=jnp.float32)
                  - pw + b_ref[...])


def _tc2_body(a_ref, m_ref, x_ref, h_ref, pos_ref, w1_ref, w2_ref, w3_ref,
              b_ref, ac_out, bc_out, u_out):
    m = m_ref[...]
    g = jax.nn.sigmoid(jnp.where(m > -1e37, a_ref[...] + m, 0.0))
    r = g[:, :_F_OUT]
    u_out[...] = g[:, _F_OUT:]
    xh2 = jnp.concatenate([x_ref[...], h_ref[...] * r], axis=1)
    w2 = w2_ref[...]
    pw = _pos_term(pos_ref, w3_ref[...])
    bc_out[...] = jnp.dot(xh2, w2, preferred_element_type=jnp.float32) + pw
    ac_out[...] = (jnp.dot(xh2, w1_ref[...] - w2, preferred_element_type=jnp.float32)
                   - pw + b_ref[...])


def _tc3_body(a_ref, m_ref, u_ref, h_ref, o_ref):
    m = m_ref[...]
    ht = jnp.tanh(jnp.where(m > -1e37, a_ref[...] + m, 0.0))
    u = u_ref[...]
    o_ref[...] = (1.0 - u) * h_ref[...] + u * ht


def _row_spec(w):
    return pl.BlockSpec((_BLK, w), lambda i: (i, 0))


def _full_spec(r, c):
    return pl.BlockSpec((r, c), lambda i: (0, 0))


def _tc1(xh, pos, w1, w2, w3, b):
    f = w1.shape[1]
    return pl.pallas_call(
        _tc1_body,
        grid=(_GRID,),
        in_specs=[_row_spec(2 * _F_IN), _row_spec(3), _full_spec(2 * _F_IN, f),
                  _full_spec(2 * _F_IN, f), _full_spec(3, f), _full_spec(1, f)],
        out_specs=[_row_spec(f), _row_spec(f)],
        out_shape=[jax.ShapeDtypeStruct((_N, f), jnp.float32)] * 2,
    )(xh, pos, w1, w2, w3, b)


def _tc2(ag, mg, x, h, pos, w1, w2, w3, b):
    f = w1.shape[1]
    return pl.pallas_call(
        _tc2_body,
        grid=(_GRID,),
        in_specs=[_row_spec(2 * _F_OUT), _row_spec(2 * _F_OUT), _row_spec(_F_IN),
                  _row_spec(_F_OUT), _row_spec(3), _full_spec(2 * _F_IN, f),
                  _full_spec(2 * _F_IN, f), _full_spec(3, f), _full_spec(1, f)],
        out_specs=[_row_spec(f), _row_spec(f), _row_spec(_F_OUT)],
        out_shape=[jax.ShapeDtypeStruct((_N, f), jnp.float32)] * 2
        + [jax.ShapeDtypeStruct((_N, _F_OUT), jnp.float32)],
    )(ag, mg, x, h, pos, w1, w2, w3, b)


def _tc3(ac, mc, u, h):
    return pl.pallas_call(
        _tc3_body,
        grid=(_GRID,),
        in_specs=[_row_spec(_F_OUT)] * 4,
        out_specs=_row_spec(_F_OUT),
        out_shape=jax.ShapeDtypeStruct((_N, _F_OUT), jnp.float32),
    )(ac, mc, u, h)


@jax.jit
def kernel(h, x, pos, edge_index_gate, edge_index_cand, Wg, bg, Wc, bc):
    k = 2 * _F_IN
    xh = jnp.concatenate([x, h], axis=1)
    ag, bgm = _tc1(xh, pos, Wg[:k], Wg[k:2 * k], Wg[2 * k:], bg.reshape(1, -1))
    mg = _segmax_gate(bgm, edge_index_gate[1], edge_index_gate[0])
    mg = mg.reshape(_NPAD, 2 * _F_OUT)[:_N]
    ac, bcm, u = _tc2(ag, mg, x, h, pos, Wc[:k], Wc[k:2 * k], Wc[2 * k:],
                      bc.reshape(1, -1))
    mc = _segmax_cand(bcm, edge_index_cand[1], edge_index_cand[0])
    mc = mc.reshape(_NPAD, _F_OUT)[:_N]
    return _tc3(ac, mc, u, h)


# speedup vs baseline: 1.4622x; 1.3806x over previous
import functools

import jax
import jax.numpy as jnp
from jax import lax
from jax.experimental import pallas as pl
from jax.experimental.pallas import tpu as pltpu
from jax.experimental.pallas import tpu_sc as plsc

_N = 10000
_E = 320000
_F_IN = 128
_F_OUT = 128
_NW = 32
_NPT = 313
_NPAD = _NW * _NPT
_NEG = float(jnp.finfo(jnp.float32).min)


def _make_segmax(F: int):
    C = 2000
    G = 8192 // F
    UN = 5
    NCH = _E // C
    PK = C + 2 * G + 32
    Fv = F // 16

    mesh = plsc.VectorSubcoreMesh(core_axis_name="c", subcore_axis_name="s")

    def body(b_hbm, dst_hbm, src_hbm, m_hbm, dstb, srcb, pks, pkd, rows, acc,
             esem, gsem):
        wid = lax.axis_index("s") * 2 + lax.axis_index("c")
        n0 = wid * _NPT

        def ini_acc(i, _):
            acc[pl.ds(i * 16, 16)] = jnp.full((16,), _NEG, jnp.float32)
            return 0
        lax.fori_loop(0, (_NPT + 1) * Fv, ini_acc, 0)

        def ini_pk(i, _):
            pks[pl.ds(i * 16, 16)] = jnp.zeros((16,), jnp.int32)
            return 0
        lax.fori_loop(0, PK // 16, ini_pk, 0)

        def start_edges(c):
            co = (c % 2) * C
            pltpu.async_copy(dst_hbm.at[pl.ds(c * C, C)],
                             dstb.at[pl.ds(co, C)], esem)
            pltpu.async_copy(src_hbm.at[pl.ds(c * C, C)],
                             srcb.at[pl.ds(co, C)], esem)

        def wait_edges(c):
            co = (c % 2) * C
            pltpu.make_async_copy(dst_hbm.at[pl.ds(0, C)],
                                  dstb.at[pl.ds(co, C)], esem).wait()
            pltpu.make_async_copy(src_hbm.at[pl.ds(0, C)],
                                  srcb.at[pl.ds(co, C)], esem).wait()

        def start_gather(sb):
            ro = (sb % 2) * G
            pltpu.async_copy(b_hbm.at[pks.at[pl.ds(sb * G, G)]],
                             rows.at[pl.ds(ro, G)], gsem)

        def wait_gather(sb):
            ro = (sb % 2) * G
            pltpu.make_async_copy(b_hbm.at[pl.ds(0, G)],
                                  rows.at[pl.ds(ro, G)], gsem).wait()

        start_edges(0)

        def chunk(c, _):
            co = (c % 2) * C
            wait_edges(c)

            @pl.when(c + 1 < NCH)
            def _():
                start_edges(c + 1)

            def filt(k, cnt):
                base = co + k * (16 * UN)
                for t in range(UN):
                    d = dstb[pl.ds(base + t * 16, 16)]
                    s = srcb[pl.ds(base + t * 16, 16)]
                    msk = (d >= n0) & (d < n0 + _NPT)
                    pc = plsc.cumsum(msk.astype(jnp.int32))
                    idx = cnt + pc - 1
                    plsc.store_scatter(pks, [idx], s, mask=msk)
                    plsc.store_scatter(pkd, [idx], d - n0, mask=msk)
                    cnt = cnt + plsc.all_reduce_population_count(msk)[0]
                return cnt
            cnt = lax.fori_loop(0, C // (16 * UN), filt, 0)

            for t in range(G // 16):
                pkd[pl.ds(cnt + t * 16, 16)] = jnp.full((16,), _NPT, jnp.int32)

            nsub = (cnt + G - 1) // G

            @pl.when(nsub > 0)
            def _():
                start_gather(0)

            def sub(sb, _):
                wait_gather(sb)

                @pl.when(sb + 1 < nsub)
                def _():
                    start_gather(sb + 1)

                ro = (sb % 2) * G
                for g in range(G // 16):
                    ldv = pkd[pl.ds(sb * G + g * 16, 16)]
                    for lane in range(16):
                        ao = ldv[lane] * F
                        rr = ro + g * 16 + lane
                        for j in range(Fv):
                            a = acc[pl.ds(ao + j * 16, 16)]
                            r = rows[rr, pl.ds(j * 16, 16)]
                            acc[pl.ds(ao + j * 16, 16)] = jnp.maximum(a, r)
                return 0
            lax.fori_loop(0, nsub, sub, 0)
            return 0
        lax.fori_loop(0, NCH, chunk, 0)

        pltpu.sync_copy(acc.at[pl.ds(0, _NPT * F)],
                        m_hbm.at[pl.ds(n0 * F, _NPT * F)])

    return pl.kernel(
        body,
        mesh=mesh,
        compiler_params=pltpu.CompilerParams(needs_layout_passes=False),
        out_type=jax.ShapeDtypeStruct((_NPAD * F,), jnp.float32),
        scratch_types=[
            pltpu.VMEM((2 * C,), jnp.int32),
            pltpu.VMEM((2 * C,), jnp.int32),
            pltpu.VMEM((PK,), jnp.int32),
            pltpu.VMEM((PK,), jnp.int32),
            pltpu.VMEM((2 * G, F), jnp.float32),
            pltpu.VMEM(((_NPT + 1) * F,), jnp.float32),
            pltpu.SemaphoreType.DMA,
            pltpu.SemaphoreType.DMA,
        ],
    )


_segmax_gate = _make_segmax(2 * _F_OUT)
_segmax_cand = _make_segmax(_F_OUT)


_BLK = 2000
_GRID = _N // _BLK


def _pos_term(pos_ref, w3):
    return (pos_ref[:, 0:1] * w3[0:1, :]
            + pos_ref[:, 1:2] * w3[1:2, :]
            + pos_ref[:, 2:3] * w3[2:3, :])


def _tc1_body(xh_ref, pos_ref, w1_ref, w2_ref, w3_ref, b_ref, a_out, b_out):
    xh = xh_ref[...]
    w2 = w2_ref[...]
    pw = _pos_term(pos_ref, w3_ref[...])
    b_out[...] = jnp.dot(xh, w2, preferred_element_type=jnp.float32) + pw
    a_out[...] = (jnp.dot(xh, w1_ref[...] - w2, preferred_element_type=jnp.float32)
                  - pw + b_ref[...])


def _tc2_body(a_ref, m_ref, x_ref, h_ref, pos_ref, w1_ref, w2_ref, w3_ref,
              b_ref, ac_out, bc_out, u_out):
    m = m_ref[...]
    g = jax.nn.sigmoid(jnp.where(m > -1e37, a_ref[...] + m, 0.0))
    r = g[:, :_F_OUT]
    u_out[...] = g[:, _F_OUT:]
    xh2 = jnp.concatenate([x_ref[...], h_ref[...] * r], axis=1)
    w2 = w2_ref[...]
    pw = _pos_term(pos_ref, w3_ref[...])
    bc_out[...] = jnp.dot(xh2, w2, preferred_element_type=jnp.float32) + pw
    ac_out[...] = (jnp.dot(xh2, w1_ref[...] - w2, preferred_element_type=jnp.float32)
                   - pw + b_ref[...])


def _tc3_body(a_ref, m_ref, u_ref, h_ref, o_ref):
    m = m_ref[...]
    ht = jnp.tanh(jnp.where(m > -1e37, a_ref[...] + m, 0.0))
    u = u_ref[...]
    o_ref[...] = (1.0 - u) * h_ref[...] + u * ht


def _row_spec(w):
    return pl.BlockSpec((_BLK, w), lambda i: (i, 0))


def _full_spec(r, c):
    return pl.BlockSpec((r, c), lambda i: (0, 0))


def _tc1(xh, pos, w1, w2, w3, b):
    f = w1.shape[1]
    return pl.pallas_call(
        _tc1_body,
        grid=(_GRID,),
        in_specs=[_row_spec(2 * _F_IN), _row_spec(3), _full_spec(2 * _F_IN, f),
                  _full_spec(2 * _F_IN, f), _full_spec(3, f), _full_spec(1, f)],
        out_specs=[_row_spec(f), _row_spec(f)],
        out_shape=[jax.ShapeDtypeStruct((_N, f), jnp.float32)] * 2,
    )(xh, pos, w1, w2, w3, b)


def _tc2(ag, mg, x, h, pos, w1, w2, w3, b):
    f = w1.shape[1]
    return pl.pallas_call(
        _tc2_body,
        grid=(_GRID,),
        in_specs=[_row_spec(2 * _F_OUT), _row_spec(2 * _F_OUT), _row_spec(_F_IN),
                  _row_spec(_F_OUT), _row_spec(3), _full_spec(2 * _F_IN, f),
                  _full_spec(2 * _F_IN, f), _full_spec(3, f), _full_spec(1, f)],
        out_specs=[_row_spec(f), _row_spec(f), _row_spec(_F_OUT)],
        out_shape=[jax.ShapeDtypeStruct((_N, f), jnp.float32)] * 2
        + [jax.ShapeDtypeStruct((_N, _F_OUT), jnp.float32)],
    )(ag, mg, x, h, pos, w1, w2, w3, b)


def _tc3(ac, mc, u, h):
    return pl.pallas_call(
        _tc3_body,
        grid=(_GRID,),
        in_specs=[_row_spec(_F_OUT)] * 4,
        out_specs=_row_spec(_F_OUT),
        out_shape=jax.ShapeDtypeStruct((_N, _F_OUT), jnp.float32),
    )(ac, mc, u, h)


@jax.jit
def kernel(h, x, pos, edge_index_gate, edge_index_cand, Wg, bg, Wc, bc):
    k = 2 * _F_IN
    xh = jnp.concatenate([x, h], axis=1)
    ag, bgm = _tc1(xh, pos, Wg[:k], Wg[k:2 * k], Wg[2 * k:], bg.reshape(1, -1))
    mg = _segmax_gate(bgm, edge_index_gate[1], edge_index_gate[0])
    mg = mg.reshape(_NPAD, 2 * _F_OUT)[:_N]
    ac, bcm, u = _tc2(ag, mg, x, h, pos, Wc[:k], Wc[k:2 * k], Wc[2 * k:],
                      bc.reshape(1, -1))
    mc = _segmax_cand(bcm, edge_index_cand[1], edge_index_cand[0])
    mc = mc.reshape(_NPAD, _F_OUT)[:_N]
    return _tc3(ac, mc, u, h)


# speedup vs baseline: 1.5616x; 1.0680x over previous
import functools

import jax
import jax.numpy as jnp
from jax import lax
from jax.experimental import pallas as pl
from jax.experimental.pallas import tpu as pltpu
from jax.experimental.pallas import tpu_sc as plsc

_N = 10000
_E = 320000
_F_IN = 128
_F_OUT = 128
_NW = 32
_NPT = 313
_NPAD = _NW * _NPT
_NEG = float(jnp.finfo(jnp.float32).min)


def _make_segmax(F: int):
    C = 2000
    G = 8192 // F
    UN = 5
    NCH = _E // C
    PK = C + 2 * G + 32
    Fv = F // 16

    mesh = plsc.VectorSubcoreMesh(core_axis_name="c", subcore_axis_name="s")

    def body(b_hbm, dst_hbm, src_hbm, m_hbm, dstb, srcb, pks, pkd, rows, acc,
             esem, gsem):
        wid = lax.axis_index("s") * 2 + lax.axis_index("c")
        n0 = wid * _NPT

        def ini_acc(i, _):
            acc[pl.ds(i * 16, 16)] = jnp.full((16,), _NEG, jnp.float32)
            return 0
        lax.fori_loop(0, (_NPT + 1) * Fv, ini_acc, 0)

        def ini_pk(i, _):
            pks[pl.ds(i * 16, 16)] = jnp.zeros((16,), jnp.int32)
            return 0
        lax.fori_loop(0, PK // 16, ini_pk, 0)

        def start_edges(c):
            co = (c % 2) * C
            pltpu.async_copy(dst_hbm.at[pl.ds(c * C, C)],
                             dstb.at[pl.ds(co, C)], esem)
            pltpu.async_copy(src_hbm.at[pl.ds(c * C, C)],
                             srcb.at[pl.ds(co, C)], esem)

        def wait_edges(c):
            co = (c % 2) * C
            pltpu.make_async_copy(dst_hbm.at[pl.ds(0, C)],
                                  dstb.at[pl.ds(co, C)], esem).wait()
            pltpu.make_async_copy(src_hbm.at[pl.ds(0, C)],
                                  srcb.at[pl.ds(co, C)], esem).wait()

        def start_gather(sb):
            ro = (sb % 2) * G
            pltpu.async_copy(b_hbm.at[pks.at[pl.ds(sb * G, G)]],
                             rows.at[pl.ds(ro, G)], gsem)

        def wait_gather(sb):
            ro = (sb % 2) * G
            pltpu.make_async_copy(b_hbm.at[pl.ds(0, G)],
                                  rows.at[pl.ds(ro, G)], gsem).wait()

        start_edges(0)

        def chunk(c, _):
            co = (c % 2) * C
            wait_edges(c)

            @pl.when(c + 1 < NCH)
            def _():
                start_edges(c + 1)

            def filt(k, cnt):
                base = co + k * (16 * UN)
                for t in range(UN):
                    d = dstb[pl.ds(base + t * 16, 16)]
                    s = srcb[pl.ds(base + t * 16, 16)]
                    msk = (d >= n0) & (d < n0 + _NPT)
                    pc = plsc.cumsum(msk.astype(jnp.int32))
                    idx = cnt + pc - 1
                    plsc.store_scatter(pks, [idx], s, mask=msk)
                    plsc.store_scatter(pkd, [idx], d - n0, mask=msk)
                    cnt = cnt + plsc.all_reduce_population_count(msk)[0]
                return cnt
            cnt = lax.fori_loop(0, C // (16 * UN), filt, 0)

            for t in range(G // 16):
                pkd[pl.ds(cnt + t * 16, 16)] = jnp.full((16,), _NPT, jnp.int32)

            nsub = (cnt + G - 1) // G

            @pl.when(nsub > 0)
            def _():
                start_gather(0)

            def sub(sb, _):
                wait_gather(sb)

                @pl.when(sb + 1 < nsub)
                def _():
                    start_gather(sb + 1)

                ro = (sb % 2) * G
                for g in range(G // 16):
                    ldv = pkd[pl.ds(sb * G + g * 16, 16)]
                    for lane in range(16):
                        ao = ldv[lane] * F
                        rr = ro + g * 16 + lane
                        av = [acc[pl.ds(ao + j * 16, 16)] for j in range(Fv)]
                        rv = [rows[rr, pl.ds(j * 16, 16)] for j in range(Fv)]
                        mx = [jnp.maximum(a, r) for a, r in zip(av, rv)]
                        for j in range(Fv):
                            acc[pl.ds(ao + j * 16, 16)] = mx[j]
                return 0
            lax.fori_loop(0, nsub, sub, 0)
            return 0
        lax.fori_loop(0, NCH, chunk, 0)

        pltpu.sync_copy(acc.at[pl.ds(0, _NPT * F)],
                        m_hbm.at[pl.ds(n0 * F, _NPT * F)])

    return pl.kernel(
        body,
        mesh=mesh,
        compiler_params=pltpu.CompilerParams(needs_layout_passes=False),
        out_type=jax.ShapeDtypeStruct((_NPAD * F,), jnp.float32),
        scratch_types=[
            pltpu.VMEM((2 * C,), jnp.int32),
            pltpu.VMEM((2 * C,), jnp.int32),
            pltpu.VMEM((PK,), jnp.int32),
            pltpu.VMEM((PK,), jnp.int32),
            pltpu.VMEM((2 * G, F), jnp.float32),
            pltpu.VMEM(((_NPT + 1) * F,), jnp.float32),
            pltpu.SemaphoreType.DMA,
            pltpu.SemaphoreType.DMA,
        ],
    )


_segmax_gate = _make_segmax(2 * _F_OUT)
_segmax_cand = _make_segmax(_F_OUT)


_BLK = 2000
_GRID = _N // _BLK


def _pos_term(pos_ref, w3):
    return (pos_ref[:, 0:1] * w3[0:1, :]
            + pos_ref[:, 1:2] * w3[1:2, :]
            + pos_ref[:, 2:3] * w3[2:3, :])


def _tc1_body(xh_ref, pos_ref, w1_ref, w2_ref, w3_ref, b_ref, a_out, b_out):
    xh = xh_ref[...]
    w2 = w2_ref[...]
    pw = _pos_term(pos_ref, w3_ref[...])
    b_out[...] = jnp.dot(xh, w2, preferred_element_type=jnp.float32) + pw
    a_out[...] = (jnp.dot(xh, w1_ref[...] - w2, preferred_element_type=jnp.float32)
                  - pw + b_ref[...])


def _tc2_body(a_ref, m_ref, x_ref, h_ref, pos_ref, w1_ref, w2_ref, w3_ref,
              b_ref, ac_out, bc_out, u_out):
    m = m_ref[...]
    g = jax.nn.sigmoid(jnp.where(m > -1e37, a_ref[...] + m, 0.0))
    r = g[:, :_F_OUT]
    u_out[...] = g[:, _F_OUT:]
    xh2 = jnp.concatenate([x_ref[...], h_ref[...] * r], axis=1)
    w2 = w2_ref[...]
    pw = _pos_term(pos_ref, w3_ref[...])
    bc_out[...] = jnp.dot(xh2, w2, preferred_element_type=jnp.float32) + pw
    ac_out[...] = (jnp.dot(xh2, w1_ref[...] - w2, preferred_element_type=jnp.float32)
                   - pw + b_ref[...])


def _tc3_body(a_ref, m_ref, u_ref, h_ref, o_ref):
    m = m_ref[...]
    ht = jnp.tanh(jnp.where(m > -1e37, a_ref[...] + m, 0.0))
    u = u_ref[...]
    o_ref[...] = (1.0 - u) * h_ref[...] + u * ht


def _row_spec(w):
    return pl.BlockSpec((_BLK, w), lambda i: (i, 0))


def _full_spec(r, c):
    return pl.BlockSpec((r, c), lambda i: (0, 0))


def _tc1(xh, pos, w1, w2, w3, b):
    f = w1.shape[1]
    return pl.pallas_call(
        _tc1_body,
        grid=(_GRID,),
        in_specs=[_row_spec(2 * _F_IN), _row_spec(3), _full_spec(2 * _F_IN, f),
                  _full_spec(2 * _F_IN, f), _full_spec(3, f), _full_spec(1, f)],
        out_specs=[_row_spec(f), _row_spec(f)],
        out_shape=[jax.ShapeDtypeStruct((_N, f), jnp.float32)] * 2,
    )(xh, pos, w1, w2, w3, b)


def _tc2(ag, mg, x, h, pos, w1, w2, w3, b):
    f = w1.shape[1]
    return pl.pallas_call(
        _tc2_body,
        grid=(_GRID,),
        in_specs=[_row_spec(2 * _F_OUT), _row_spec(2 * _F_OUT), _row_spec(_F_IN),
                  _row_spec(_F_OUT), _row_spec(3), _full_spec(2 * _F_IN, f),
                  _full_spec(2 * _F_IN, f), _full_spec(3, f), _full_spec(1, f)],
        out_specs=[_row_spec(f), _row_spec(f), _row_spec(_F_OUT)],
        out_shape=[jax.ShapeDtypeStruct((_N, f), jnp.float32)] * 2
        + [jax.ShapeDtypeStruct((_N, _F_OUT), jnp.float32)],
    )(ag, mg, x, h, pos, w1, w2, w3, b)


def _tc3(ac, mc, u, h):
    return pl.pallas_call(
        _tc3_body,
        grid=(_GRID,),
        in_specs=[_row_spec(_F_OUT)] * 4,
        out_specs=_row_spec(_F_OUT),
        out_shape=jax.ShapeDtypeStruct((_N, _F_OUT), jnp.float32),
    )(ac, mc, u, h)


@jax.jit
def kernel(h, x, pos, edge_index_gate, edge_index_cand, Wg, bg, Wc, bc):
    k = 2 * _F_IN
    xh = jnp.concatenate([x, h], axis=1)
    ag, bgm = _tc1(xh, pos, Wg[:k], Wg[k:2 * k], Wg[2 * k:], bg.reshape(1, -1))
    mg = _segmax_gate(bgm, edge_index_gate[1], edge_index_gate[0])
    mg = mg.reshape(_NPAD, 2 * _F_OUT)[:_N]
    ac, bcm, u = _tc2(ag, mg, x, h, pos, Wc[:k], Wc[k:2 * k], Wc[2 * k:],
                      bc.reshape(1, -1))
    mc = _segmax_cand(bcm, edge_index_cand[1], edge_index_cand[0])
    mc = mc.reshape(_NPAD, _F_OUT)[:_N]
    return _tc3(ac, mc, u, h)


# speedup vs baseline: 6.1751x; 3.9543x over previous
import functools

import jax
import jax.numpy as jnp
from jax import lax
from jax.experimental import pallas as pl
from jax.experimental.pallas import tpu as pltpu
from jax.experimental.pallas import tpu_sc as plsc

_N = 10000
_E = 320000
_F_IN = 128
_F_OUT = 128
_NW = 32
_NPT = 313
_NPAD = _NW * _NPT
_NEG = float(jnp.finfo(jnp.float32).min)


def _make_segmax(F: int):
    C = 3200
    G = 16
    R = 6 if F == 256 else 8
    UN = 5
    NCH = _E // C
    PK = C + 2 * G + 32
    Fv = F // 16

    mesh = plsc.VectorSubcoreMesh(core_axis_name="c", subcore_axis_name="s")

    def body(b_hbm, dst_hbm, src_hbm, m_hbm, dstb, srcb, pks, pkd, rows, acc,
             esem, gsem):
        wid = lax.axis_index("s") * 2 + lax.axis_index("c")
        n0 = wid * _NPT

        def ini_acc(i, _):
            acc[pl.ds(i * 16, 16)] = jnp.full((16,), _NEG, jnp.float32)
            return 0
        lax.fori_loop(0, (_NPT + 1) * Fv, ini_acc, 0)

        def ini_pk(i, _):
            pks[pl.ds(i * 16, 16)] = jnp.zeros((16,), jnp.int32)
            return 0
        lax.fori_loop(0, PK // 16, ini_pk, 0)

        def start_edges(c):
            co = (c % 2) * C
            pltpu.async_copy(dst_hbm.at[pl.ds(c * C, C)],
                             dstb.at[pl.ds(co, C)], esem)
            pltpu.async_copy(src_hbm.at[pl.ds(c * C, C)],
                             srcb.at[pl.ds(co, C)], esem)

        def wait_edges(c):
            co = (c % 2) * C
            pltpu.make_async_copy(dst_hbm.at[pl.ds(0, C)],
                                  dstb.at[pl.ds(co, C)], esem).wait()
            pltpu.make_async_copy(src_hbm.at[pl.ds(0, C)],
                                  srcb.at[pl.ds(co, C)], esem).wait()

        def start_gather(sb):
            slot = sb % R
            pltpu.async_copy(b_hbm.at[pks.at[pl.ds(sb * G, G)]],
                             rows.at[pl.ds(slot * G, G)], gsem.at[slot])

        def wait_gather(sb):
            slot = sb % R
            pltpu.make_async_copy(b_hbm.at[pl.ds(0, G)],
                                  rows.at[pl.ds(slot * G, G)],
                                  gsem.at[slot]).wait()

        start_edges(0)

        def chunk(c, _):
            co = (c % 2) * C
            wait_edges(c)

            @pl.when(c + 1 < NCH)
            def _():
                start_edges(c + 1)

            def filt(k, cnt):
                base = co + k * (16 * UN)
                for t in range(UN):
                    d = dstb[pl.ds(base + t * 16, 16)]
                    s = srcb[pl.ds(base + t * 16, 16)]
                    msk = (d >= n0) & (d < n0 + _NPT)
                    pc = plsc.cumsum(msk.astype(jnp.int32))
                    idx = cnt + pc - 1
                    plsc.store_scatter(pks, [idx], s, mask=msk)
                    plsc.store_scatter(pkd, [idx], d - n0, mask=msk)
                    cnt = cnt + plsc.all_reduce_population_count(msk)[0]
                return cnt
            cnt = lax.fori_loop(0, C // (16 * UN), filt, 0)

            for t in range(G // 16):
                pkd[pl.ds(cnt + t * 16, 16)] = jnp.full((16,), _NPT, jnp.int32)

            nsub = (cnt + G - 1) // G

            def prime(p, _):
                start_gather(p)
                return 0
            lax.fori_loop(0, jnp.minimum(nsub, R), prime, 0)

            def sub(sb, _):
                wait_gather(sb)

                @pl.when(sb + R < nsub)
                def _():
                    start_gather(sb + R)

                ro = (sb % R) * G
                ldv = pkd[pl.ds(sb * G, 16)]
                for lane in range(16):
                    ao = ldv[lane] * F
                    rr = ro + lane
                    av = [acc[pl.ds(ao + j * 16, 16)] for j in range(Fv)]
                    rv = [rows[rr, pl.ds(j * 16, 16)] for j in range(Fv)]
                    mx = [jnp.maximum(a, r) for a, r in zip(av, rv)]
                    for j in range(Fv):
                        acc[pl.ds(ao + j * 16, 16)] = mx[j]
                return 0
            lax.fori_loop(0, nsub, sub, 0)
            return 0
        lax.fori_loop(0, NCH, chunk, 0)

        pltpu.sync_copy(acc.at[pl.ds(0, _NPT * F)],
                        m_hbm.at[pl.ds(n0 * F, _NPT * F)])

    return pl.kernel(
        body,
        mesh=mesh,
        compiler_params=pltpu.CompilerParams(needs_layout_passes=False),
        out_type=jax.ShapeDtypeStruct((_NPAD * F,), jnp.float32),
        scratch_types=[
            pltpu.VMEM((2 * C,), jnp.int32),
            pltpu.VMEM((2 * C,), jnp.int32),
            pltpu.VMEM((PK,), jnp.int32),
            pltpu.VMEM((PK,), jnp.int32),
            pltpu.VMEM((R * G, F), jnp.float32),
            pltpu.VMEM(((_NPT + 1) * F,), jnp.float32),
            pltpu.SemaphoreType.DMA,
            pltpu.SemaphoreType.DMA((R,)),
        ],
    )


_segmax_gate = _make_segmax(2 * _F_OUT)
_segmax_cand = _make_segmax(_F_OUT)


_BLK = 2000
_GRID = _N // _BLK


def _pos_term(pos_ref, w3):
    return (pos_ref[:, 0:1] * w3[0:1, :]
            + pos_ref[:, 1:2] * w3[1:2, :]
            + pos_ref[:, 2:3] * w3[2:3, :])


def _tc1_body(xh_ref, pos_ref, w1_ref, w2_ref, w3_ref, b_ref, a_out, b_out):
    xh = xh_ref[...]
    w2 = w2_ref[...]
    pw = _pos_term(pos_ref, w3_ref[...])
    b_out[...] = jnp.dot(xh, w2, preferred_element_type=jnp.float32) + pw
    a_out[...] = (jnp.dot(xh, w1_ref[...] - w2, preferred_element_type=jnp.float32)
                  - pw + b_ref[...])


def _tc2_body(a_ref, m_ref, x_ref, h_ref, pos_ref, w1_ref, w2_ref, w3_ref,
              b_ref, ac_out, bc_out, u_out):
    m = m_ref[...]
    g = jax.nn.sigmoid(jnp.where(m > -1e37, a_ref[...] + m, 0.0))
    r = g[:, :_F_OUT]
    u_out[...] = g[:, _F_OUT:]
    xh2 = jnp.concatenate([x_ref[...], h_ref[...] * r], axis=1)
    w2 = w2_ref[...]
    pw = _pos_term(pos_ref, w3_ref[...])
    bc_out[...] = jnp.dot(xh2, w2, preferred_element_type=jnp.float32) + pw
    ac_out[...] = (jnp.dot(xh2, w1_ref[...] - w2, preferred_element_type=jnp.float32)
                   - pw + b_ref[...])


def _tc3_body(a_ref, m_ref, u_ref, h_ref, o_ref):
    m = m_ref[...]
    ht = jnp.tanh(jnp.where(m > -1e37, a_ref[...] + m, 0.0))
    u = u_ref[...]
    o_ref[...] = (1.0 - u) * h_ref[...] + u * ht


def _row_spec(w):
    return pl.BlockSpec((_BLK, w), lambda i: (i, 0))


def _full_spec(r, c):
    return pl.BlockSpec((r, c), lambda i: (0, 0))


def _tc1(xh, pos, w1, w2, w3, b):
    f = w1.shape[1]
    return pl.pallas_call(
        _tc1_body,
        grid=(_GRID,),
        in_specs=[_row_spec(2 * _F_IN), _row_spec(3), _full_spec(2 * _F_IN, f),
                  _full_spec(2 * _F_IN, f), _full_spec(3, f), _full_spec(1, f)],
        out_specs=[_row_spec(f), _row_spec(f)],
        out_shape=[jax.ShapeDtypeStruct((_N, f), jnp.float32)] * 2,
    )(xh, pos, w1, w2, w3, b)


def _tc2(ag, mg, x, h, pos, w1, w2, w3, b):
    f = w1.shape[1]
    return pl.pallas_call(
        _tc2_body,
        grid=(_GRID,),
        in_specs=[_row_spec(2 * _F_OUT), _row_spec(2 * _F_OUT), _row_spec(_F_IN),
                  _row_spec(_F_OUT), _row_spec(3), _full_spec(2 * _F_IN, f),
                  _full_spec(2 * _F_IN, f), _full_spec(3, f), _full_spec(1, f)],
        out_specs=[_row_spec(f), _row_spec(f), _row_spec(_F_OUT)],
        out_shape=[jax.ShapeDtypeStruct((_N, f), jnp.float32)] * 2
        + [jax.ShapeDtypeStruct((_N, _F_OUT), jnp.float32)],
    )(ag, mg, x, h, pos, w1, w2, w3, b)


def _tc3(ac, mc, u, h):
    return pl.pallas_call(
        _tc3_body,
        grid=(_GRID,),
        in_specs=[_row_spec(_F_OUT)] * 4,
        out_specs=_row_spec(_F_OUT),
        out_shape=jax.ShapeDtypeStruct((_N, _F_OUT), jnp.float32),
    )(ac, mc, u, h)


@jax.jit
def kernel(h, x, pos, edge_index_gate, edge_index_cand, Wg, bg, Wc, bc):
    k = 2 * _F_IN
    xh = jnp.concatenate([x, h], axis=1)
    ag, bgm = _tc1(xh, pos, Wg[:k], Wg[k:2 * k], Wg[2 * k:], bg.reshape(1, -1))
    mg = _segmax_gate(bgm, edge_index_gate[1], edge_index_gate[0])
    mg = mg.reshape(_NPAD, 2 * _F_OUT)[:_N]
    ac, bcm, u = _tc2(ag, mg, x, h, pos, Wc[:k], Wc[k:2 * k], Wc[2 * k:],
                      bc.reshape(1, -1))
    mc = _segmax_cand(bcm, edge_index_cand[1], edge_index_cand[0])
    mc = mc.reshape(_NPAD, _F_OUT)[:_N]
    return _tc3(ac, mc, u, h)


# speedup vs baseline: 6.2100x; 1.0057x over previous
import functools

import jax
import jax.numpy as jnp
from jax import lax
from jax.experimental import pallas as pl
from jax.experimental.pallas import tpu as pltpu
from jax.experimental.pallas import tpu_sc as plsc

_N = 10000
_E = 320000
_F_IN = 128
_F_OUT = 128
_NW = 32
_NPT = 313
_NPAD = _NW * _NPT
_NEG = float(jnp.finfo(jnp.float32).min)


def _make_segmax(F: int):
    C = 3200
    G = 16
    R = 6 if F == 256 else 8
    UN = 8
    NCH = _E // C
    PK = C + 2 * G + 32
    Fv = F // 16

    mesh = plsc.VectorSubcoreMesh(core_axis_name="c", subcore_axis_name="s")

    def body(b_hbm, dst_hbm, src_hbm, m_hbm, dstb, srcb, pks, pkd, rows, acc,
             esem, gsem):
        wid = lax.axis_index("s") * 2 + lax.axis_index("c")
        n0 = wid * _NPT

        def ini_acc(i, _):
            acc[pl.ds(i * 16, 16)] = jnp.full((16,), _NEG, jnp.float32)
            return 0
        lax.fori_loop(0, (_NPT + 1) * Fv, ini_acc, 0)

        def ini_pk(i, _):
            pks[pl.ds(i * 16, 16)] = jnp.zeros((16,), jnp.int32)
            return 0
        lax.fori_loop(0, PK // 16, ini_pk, 0)

        def start_edges(c):
            co = (c % 2) * C
            pltpu.async_copy(dst_hbm.at[pl.ds(c * C, C)],
                             dstb.at[pl.ds(co, C)], esem)
            pltpu.async_copy(src_hbm.at[pl.ds(c * C, C)],
                             srcb.at[pl.ds(co, C)], esem)

        def wait_edges(c):
            co = (c % 2) * C
            pltpu.make_async_copy(dst_hbm.at[pl.ds(0, C)],
                                  dstb.at[pl.ds(co, C)], esem).wait()
            pltpu.make_async_copy(src_hbm.at[pl.ds(0, C)],
                                  srcb.at[pl.ds(co, C)], esem).wait()

        def start_gather(sb):
            slot = sb % R
            pltpu.async_copy(b_hbm.at[pks.at[pl.ds(sb * G, G)]],
                             rows.at[pl.ds(slot * G, G)], gsem.at[slot])

        def wait_gather(sb):
            slot = sb % R
            pltpu.make_async_copy(b_hbm.at[pl.ds(0, G)],
                                  rows.at[pl.ds(slot * G, G)],
                                  gsem.at[slot]).wait()

        start_edges(0)

        def chunk(c, _):
            co = (c % 2) * C
            wait_edges(c)

            @pl.when(c + 1 < NCH)
            def _():
                start_edges(c + 1)

            fifteen = jnp.full((16,), 15, jnp.int32)

            def filt(k, cntv):
                base = co + k * (16 * UN)
                for t in range(UN):
                    d = dstb[pl.ds(base + t * 16, 16)]
                    s = srcb[pl.ds(base + t * 16, 16)]
                    msk = (d >= n0) & (d < n0 + _NPT)
                    pc = plsc.cumsum(msk.astype(jnp.int32))
                    idx = cntv + pc - 1
                    plsc.store_scatter(pks, [idx], s, mask=msk)
                    plsc.store_scatter(pkd, [idx], d - n0, mask=msk)
                    cntv = cntv + pc.at[fifteen].get(mode="promise_in_bounds")
                return cntv
            cntv = lax.fori_loop(0, C // (16 * UN), filt,
                                 jnp.zeros((16,), jnp.int32))
            cnt = cntv[0]

            for t in range(G // 16):
                pkd[pl.ds(cnt + t * 16, 16)] = jnp.full((16,), _NPT, jnp.int32)

            nsub = (cnt + G - 1) // G

            def prime(p, _):
                start_gather(p)
                return 0
            lax.fori_loop(0, jnp.minimum(nsub, R), prime, 0)

            def sub(sb, _):
                wait_gather(sb)

                @pl.when(sb + R < nsub)
                def _():
                    start_gather(sb + R)

                ro = (sb % R) * G
                ldv = pkd[pl.ds(sb * G, 16)]
                for lane in range(16):
                    ao = ldv[lane] * F
                    rr = ro + lane
                    av = [acc[pl.ds(ao + j * 16, 16)] for j in range(Fv)]
                    rv = [rows[rr, pl.ds(j * 16, 16)] for j in range(Fv)]
                    mx = [jnp.maximum(a, r) for a, r in zip(av, rv)]
                    for j in range(Fv):
                        acc[pl.ds(ao + j * 16, 16)] = mx[j]
                return 0
            lax.fori_loop(0, nsub, sub, 0)
            return 0
        lax.fori_loop(0, NCH, chunk, 0)

        pltpu.sync_copy(acc.at[pl.ds(0, _NPT * F)],
                        m_hbm.at[pl.ds(n0 * F, _NPT * F)])

    return pl.kernel(
        body,
        mesh=mesh,
        compiler_params=pltpu.CompilerParams(needs_layout_passes=False),
        out_type=jax.ShapeDtypeStruct((_NPAD * F,), jnp.float32),
        scratch_types=[
            pltpu.VMEM((2 * C,), jnp.int32),
            pltpu.VMEM((2 * C,), jnp.int32),
            pltpu.VMEM((PK,), jnp.int32),
            pltpu.VMEM((PK,), jnp.int32),
            pltpu.VMEM((R * G, F), jnp.float32),
            pltpu.VMEM(((_NPT + 1) * F,), jnp.float32),
            pltpu.SemaphoreType.DMA,
            pltpu.SemaphoreType.DMA((R,)),
        ],
    )


_segmax_gate = _make_segmax(2 * _F_OUT)
_segmax_cand = _make_segmax(_F_OUT)


_BLK = 2000
_GRID = _N // _BLK


def _pos_term(pos_ref, w3):
    return (pos_ref[:, 0:1] * w3[0:1, :]
            + pos_ref[:, 1:2] * w3[1:2, :]
            + pos_ref[:, 2:3] * w3[2:3, :])


def _tc1_body(xh_ref, pos_ref, w1_ref, w2_ref, w3_ref, b_ref, a_out, b_out):
    xh = xh_ref[...]
    w2 = w2_ref[...]
    pw = _pos_term(pos_ref, w3_ref[...])
    b_out[...] = jnp.dot(xh, w2, preferred_element_type=jnp.float32) + pw
    a_out[...] = (jnp.dot(xh, w1_ref[...] - w2, preferred_element_type=jnp.float32)
                  - pw + b_ref[...])


def _tc2_body(a_ref, m_ref, x_ref, h_ref, pos_ref, w1_ref, w2_ref, w3_ref,
              b_ref, ac_out, bc_out, u_out):
    m = m_ref[...]
    g = jax.nn.sigmoid(jnp.where(m > -1e37, a_ref[...] + m, 0.0))
    r = g[:, :_F_OUT]
    u_out[...] = g[:, _F_OUT:]
    xh2 = jnp.concatenate([x_ref[...], h_ref[...] * r], axis=1)
    w2 = w2_ref[...]
    pw = _pos_term(pos_ref, w3_ref[...])
    bc_out[...] = jnp.dot(xh2, w2, preferred_element_type=jnp.float32) + pw
    ac_out[...] = (jnp.dot(xh2, w1_ref[...] - w2, preferred_element_type=jnp.float32)
                   - pw + b_ref[...])


def _tc3_body(a_ref, m_ref, u_ref, h_ref, o_ref):
    m = m_ref[...]
    ht = jnp.tanh(jnp.where(m > -1e37, a_ref[...] + m, 0.0))
    u = u_ref[...]
    o_ref[...] = (1.0 - u) * h_ref[...] + u * ht


def _row_spec(w):
    return pl.BlockSpec((_BLK, w), lambda i: (i, 0))


def _full_spec(r, c):
    return pl.BlockSpec((r, c), lambda i: (0, 0))


def _tc1(xh, pos, w1, w2, w3, b):
    f = w1.shape[1]
    return pl.pallas_call(
        _tc1_body,
        grid=(_GRID,),
        in_specs=[_row_spec(2 * _F_IN), _row_spec(3), _full_spec(2 * _F_IN, f),
                  _full_spec(2 * _F_IN, f), _full_spec(3, f), _full_spec(1, f)],
        out_specs=[_row_spec(f), _row_spec(f)],
        out_shape=[jax.ShapeDtypeStruct((_N, f), jnp.float32)] * 2,
    )(xh, pos, w1, w2, w3, b)


def _tc2(ag, mg, x, h, pos, w1, w2, w3, b):
    f = w1.shape[1]
    return pl.pallas_call(
        _tc2_body,
        grid=(_GRID,),
        in_specs=[_row_spec(2 * _F_OUT), _row_spec(2 * _F_OUT), _row_spec(_F_IN),
                  _row_spec(_F_OUT), _row_spec(3), _full_spec(2 * _F_IN, f),
                  _full_spec(2 * _F_IN, f), _full_spec(3, f), _full_spec(1, f)],
        out_specs=[_row_spec(f), _row_spec(f), _row_spec(_F_OUT)],
        out_shape=[jax.ShapeDtypeStruct((_N, f), jnp.float32)] * 2
        + [jax.ShapeDtypeStruct((_N, _F_OUT), jnp.float32)],
    )(ag, mg, x, h, pos, w1, w2, w3, b)


def _tc3(ac, mc, u, h):
    return pl.pallas_call(
        _tc3_body,
        grid=(_GRID,),
        in_specs=[_row_spec(_F_OUT)] * 4,
        out_specs=_row_spec(_F_OUT),
        out_shape=jax.ShapeDtypeStruct((_N, _F_OUT), jnp.float32),
    )(ac, mc, u, h)


@jax.jit
def kernel(h, x, pos, edge_index_gate, edge_index_cand, Wg, bg, Wc, bc):
    k = 2 * _F_IN
    xh = jnp.concatenate([x, h], axis=1)
    ag, bgm = _tc1(xh, pos, Wg[:k], Wg[k:2 * k], Wg[2 * k:], bg.reshape(1, -1))
    mg = _segmax_gate(bgm, edge_index_gate[1], edge_index_gate[0])
    mg = mg.reshape(_NPAD, 2 * _F_OUT)[:_N]
    ac, bcm, u = _tc2(ag, mg, x, h, pos, Wc[:k], Wc[k:2 * k], Wc[2 * k:],
                      bc.reshape(1, -1))
    mc = _segmax_cand(bcm, edge_index_cand[1], edge_index_cand[0])
    mc = mc.reshape(_NPAD, _F_OUT)[:_N]
    return _tc3(ac, mc, u, h)


# speedup vs baseline: 7.0984x; 1.1430x over previous
import functools

import jax
import jax.numpy as jnp
from jax import lax
from jax.experimental import pallas as pl
from jax.experimental.pallas import tpu as pltpu
from jax.experimental.pallas import tpu_sc as plsc

_N = 10000
_E = 320000
_F_IN = 128
_F_OUT = 128
_NW = 32
_NPT = 313
_NPAD = _NW * _NPT
_NEG = float(jnp.finfo(jnp.float32).min)


def _make_segmax(F: int):
    C = 3200
    G = 16
    R = 5 if F == 256 else 8
    UN = 8
    NCH = _E // C
    PK = C + 2 * G + 32
    Fv = F // 16

    mesh = plsc.VectorSubcoreMesh(core_axis_name="c", subcore_axis_name="s")

    def body(b_hbm, dst_hbm, src_hbm, m_hbm, dstb, srcb, pks0, pkd0, pks1,
             pkd1, rows, acc, esem, gsem):
        wid = lax.axis_index("s") * 2 + lax.axis_index("c")
        n0 = wid * _NPT
        pk = ((pks0, pkd0), (pks1, pkd1))

        def ini_acc(i, _):
            acc[pl.ds(i * 16, 16)] = jnp.full((16,), _NEG, jnp.float32)
            return 0
        lax.fori_loop(0, (_NPT + 1) * Fv, ini_acc, 0)

        def ini_pk(i, _):
            pks0[pl.ds(i * 16, 16)] = jnp.zeros((16,), jnp.int32)
            pks1[pl.ds(i * 16, 16)] = jnp.zeros((16,), jnp.int32)
            return 0
        lax.fori_loop(0, PK // 16, ini_pk, 0)

        def start_edges(c):
            co = (c % 2) * C
            pltpu.async_copy(dst_hbm.at[pl.ds(c * C, C)],
                             dstb.at[pl.ds(co, C)], esem)
            pltpu.async_copy(src_hbm.at[pl.ds(c * C, C)],
                             srcb.at[pl.ds(co, C)], esem)

        def wait_edges(c):
            co = (c % 2) * C
            pltpu.make_async_copy(dst_hbm.at[pl.ds(0, C)],
                                  dstb.at[pl.ds(co, C)], esem).wait()
            pltpu.make_async_copy(src_hbm.at[pl.ds(0, C)],
                                  srcb.at[pl.ds(co, C)], esem).wait()

        def start_gather(sb, par):
            slot = sb % R
            pltpu.async_copy(b_hbm.at[pk[par][0].at[pl.ds(sb * G, G)]],
                             rows.at[pl.ds(slot * G, G)], gsem.at[slot])

        def wait_gather(sb):
            slot = sb % R
            pltpu.make_async_copy(b_hbm.at[pl.ds(0, G)],
                                  rows.at[pl.ds(slot * G, G)],
                                  gsem.at[slot]).wait()

        fifteen = jnp.full((16,), 15, jnp.int32)

        def do_filter(c, par):
            co = par * C
            pks_r, pkd_r = pk[par]
            wait_edges(c)

            def filt(k, cntv):
                base = co + k * (16 * UN)
                for t in range(UN):
                    d = dstb[pl.ds(base + t * 16, 16)]
                    s = srcb[pl.ds(base + t * 16, 16)]
                    msk = (d >= n0) & (d < n0 + _NPT)
                    pc = plsc.cumsum(msk.astype(jnp.int32))
                    idx = cntv + pc - 1
                    plsc.store_scatter(pks_r, [idx], s, mask=msk)
                    plsc.store_scatter(pkd_r, [idx], d - n0, mask=msk)
                    cntv = cntv + pc.at[fifteen].get(mode="promise_in_bounds")
                return cntv
            cntv = lax.fori_loop(0, C // (16 * UN), filt,
                                 jnp.zeros((16,), jnp.int32))
            cnt = cntv[0]
            pkd_r[pl.ds(cnt, 16)] = jnp.full((16,), _NPT, jnp.int32)

            @pl.when(c + 2 < NCH)
            def _():
                start_edges(c + 2)
            return cnt

        def do_prime(cnt, par):
            nsub = (cnt + G - 1) // G

            def prime(p, _):
                start_gather(p, par)
                return 0
            lax.fori_loop(0, jnp.minimum(nsub, R), prime, 0)

        def do_process(cnt, par):
            nsub = (cnt + G - 1) // G
            pkd_r = pk[par][1]

            def sub(sb, _):
                wait_gather(sb)

                @pl.when(sb + R < nsub)
                def _():
                    start_gather(sb + R, par)

                ro = (sb % R) * G
                ldv = pkd_r[pl.ds(sb * G, 16)]
                for lane in range(16):
                    ao = ldv[lane] * F
                    rr = ro + lane
                    av = [acc[pl.ds(ao + j * 16, 16)] for j in range(Fv)]
                    rv = [rows[rr, pl.ds(j * 16, 16)] for j in range(Fv)]
                    mx = [jnp.maximum(a, r) for a, r in zip(av, rv)]
                    for j in range(Fv):
                        acc[pl.ds(ao + j * 16, 16)] = mx[j]
                return 0
            lax.fori_loop(0, nsub, sub, 0)

        start_edges(0)
        start_edges(1)
        cnt0 = do_filter(0, 0)
        do_prime(cnt0, 0)

        def two(i, cnt_even):
            c_odd = 2 * i + 1
            cnt_odd = do_filter(c_odd, 1)
            do_process(cnt_even, 0)
            do_prime(cnt_odd, 1)
            c_even2 = 2 * i + 2
            cnt_even2 = lax.cond(c_even2 < NCH,
                                 lambda: do_filter(c_even2, 0),
                                 lambda: 0)
            do_process(cnt_odd, 1)

            @pl.when(c_even2 < NCH)
            def _():
                do_prime(cnt_even2, 0)
            return cnt_even2
        lax.fori_loop(0, NCH // 2, two, cnt0)

        pltpu.sync_copy(acc.at[pl.ds(0, _NPT * F)],
                        m_hbm.at[pl.ds(n0 * F, _NPT * F)])

    return pl.kernel(
        body,
        mesh=mesh,
        compiler_params=pltpu.CompilerParams(needs_layout_passes=False),
        out_type=jax.ShapeDtypeStruct((_NPAD * F,), jnp.float32),
        scratch_types=[
            pltpu.VMEM((2 * C,), jnp.int32),
            pltpu.VMEM((2 * C,), jnp.int32),
            pltpu.VMEM((PK,), jnp.int32),
            pltpu.VMEM((PK,), jnp.int32),
            pltpu.VMEM((PK,), jnp.int32),
            pltpu.VMEM((PK,), jnp.int32),
            pltpu.VMEM((R * G, F), jnp.float32),
            pltpu.VMEM(((_NPT + 1) * F,), jnp.float32),
            pltpu.SemaphoreType.DMA,
            pltpu.SemaphoreType.DMA((R,)),
        ],
    )


_segmax_gate = _make_segmax(2 * _F_OUT)
_segmax_cand = _make_segmax(_F_OUT)


_BLK = 2000
_GRID = _N // _BLK


def _pos_term(pos_ref, w3):
    return (pos_ref[:, 0:1] * w3[0:1, :]
            + pos_ref[:, 1:2] * w3[1:2, :]
            + pos_ref[:, 2:3] * w3[2:3, :])


def _tc1_body(xh_ref, pos_ref, w1_ref, w2_ref, w3_ref, b_ref, a_out, b_out):
    xh = xh_ref[...]
    w2 = w2_ref[...]
    pw = _pos_term(pos_ref, w3_ref[...])
    b_out[...] = jnp.dot(xh, w2, preferred_element_type=jnp.float32) + pw
    a_out[...] = (jnp.dot(xh, w1_ref[...] - w2, preferred_element_type=jnp.float32)
                  - pw + b_ref[...])


def _tc2_body(a_ref, m_ref, x_ref, h_ref, pos_ref, w1_ref, w2_ref, w3_ref,
              b_ref, ac_out, bc_out, u_out):
    m = m_ref[...]
    g = jax.nn.sigmoid(jnp.where(m > -1e37, a_ref[...] + m, 0.0))
    r = g[:, :_F_OUT]
    u_out[...] = g[:, _F_OUT:]
    xh2 = jnp.concatenate([x_ref[...], h_ref[...] * r], axis=1)
    w2 = w2_ref[...]
    pw = _pos_term(pos_ref, w3_ref[...])
    bc_out[...] = jnp.dot(xh2, w2, preferred_element_type=jnp.float32) + pw
    ac_out[...] = (jnp.dot(xh2, w1_ref[...] - w2, preferred_element_type=jnp.float32)
                   - pw + b_ref[...])


def _tc3_body(a_ref, m_ref, u_ref, h_ref, o_ref):
    m = m_ref[...]
    ht = jnp.tanh(jnp.where(m > -1e37, a_ref[...] + m, 0.0))
    u = u_ref[...]
    o_ref[...] = (1.0 - u) * h_ref[...] + u * ht


def _row_spec(w):
    return pl.BlockSpec((_BLK, w), lambda i: (i, 0))


def _full_spec(r, c):
    return pl.BlockSpec((r, c), lambda i: (0, 0))


def _tc1(xh, pos, w1, w2, w3, b):
    f = w1.shape[1]
    return pl.pallas_call(
        _tc1_body,
        grid=(_GRID,),
        in_specs=[_row_spec(2 * _F_IN), _row_spec(3), _full_spec(2 * _F_IN, f),
                  _full_spec(2 * _F_IN, f), _full_spec(3, f), _full_spec(1, f)],
        out_specs=[_row_spec(f), _row_spec(f)],
        out_shape=[jax.ShapeDtypeStruct((_N, f), jnp.float32)] * 2,
    )(xh, pos, w1, w2, w3, b)


def _tc2(ag, mg, x, h, pos, w1, w2, w3, b):
    f = w1.shape[1]
    return pl.pallas_call(
        _tc2_body,
        grid=(_GRID,),
        in_specs=[_row_spec(2 * _F_OUT), _row_spec(2 * _F_OUT), _row_spec(_F_IN),
                  _row_spec(_F_OUT), _row_spec(3), _full_spec(2 * _F_IN, f),
                  _full_spec(2 * _F_IN, f), _full_spec(3, f), _full_spec(1, f)],
        out_specs=[_row_spec(f), _row_spec(f), _row_spec(_F_OUT)],
        out_shape=[jax.ShapeDtypeStruct((_N, f), jnp.float32)] * 2
        + [jax.ShapeDtypeStruct((_N, _F_OUT), jnp.float32)],
    )(ag, mg, x, h, pos, w1, w2, w3, b)


def _tc3(ac, mc, u, h):
    return pl.pallas_call(
        _tc3_body,
        grid=(_GRID,),
        in_specs=[_row_spec(_F_OUT)] * 4,
        out_specs=_row_spec(_F_OUT),
        out_shape=jax.ShapeDtypeStruct((_N, _F_OUT), jnp.float32),
    )(ac, mc, u, h)


@jax.jit
def kernel(h, x, pos, edge_index_gate, edge_index_cand, Wg, bg, Wc, bc):
    k = 2 * _F_IN
    xh = jnp.concatenate([x, h], axis=1)
    ag, bgm = _tc1(xh, pos, Wg[:k], Wg[k:2 * k], Wg[2 * k:], bg.reshape(1, -1))
    mg = _segmax_gate(bgm, edge_index_gate[1], edge_index_gate[0])
    mg = mg.reshape(_NPAD, 2 * _F_OUT)[:_N]
    ac, bcm, u = _tc2(ag, mg, x, h, pos, Wc[:k], Wc[k:2 * k], Wc[2 * k:],
                      bc.reshape(1, -1))
    mc = _segmax_cand(bcm, edge_index_cand[1], edge_index_cand[0])
    mc = mc.reshape(_NPAD, _F_OUT)[:_N]
    return _tc3(ac, mc, u, h)
